# baseline (speedup 1.0000x reference)
import jax
import jax.numpy as jnp
from jax.experimental import pallas as pl

B = 64
H = 4096
V = 32000
D = 8
VT = 640


def _body(x_ref, w_ref, idx_ref, o_ref):
    e = pl.program_id(1)

    @pl.when(e == 0)
    def _():
        o_ref[...] = jnp.zeros_like(o_ref)

    xb = x_ref[...]
    wb = w_ref[0].astype(jnp.bfloat16)
    part = jax.lax.dot_general(
        xb, wb, (((1,), (1,)), ((), ())),
        preferred_element_type=jnp.float32)
    mask = idx_ref[...] == e
    o_ref[...] += jnp.where(mask, part, 0.0)


def kernel(hidden_states, weight_stacked, indices):
    idx = indices.astype(jnp.int32).reshape(B, 1)
    hidden_states = hidden_states.astype(jnp.bfloat16)
    grid = (V // VT, D)
    return pl.pallas_call(
        _body,
        grid=grid,
        in_specs=[
            pl.BlockSpec((B, H), lambda v, e: (0, 0)),
            pl.BlockSpec((1, VT, H), lambda v, e: (e, v, 0)),
            pl.BlockSpec((B, 1), lambda v, e: (0, 0)),
        ],
        out_specs=pl.BlockSpec((B, VT), lambda v, e: (0, v)),
        out_shape=jax.ShapeDtypeStruct((B, V), jnp.float32),
    )(hidden_states, weight_stacked, idx)


# speedup vs baseline: 1.0083x; 1.0083x over previous
import jax
import jax.numpy as jnp
from jax.experimental import pallas as pl

B = 64
H = 4096
V = 32000
D = 8
VT = 1280


def _body(x_ref, w_ref, idx_ref, o_ref):
    e = pl.program_id(1)

    @pl.when(e == 0)
    def _():
        o_ref[...] = jnp.zeros_like(o_ref)

    xb = x_ref[...]
    wb = w_ref[0]
    part = jax.lax.dot_general(
        xb, wb, (((1,), (1,)), ((), ())),
        preferred_element_type=jnp.float32)
    mask = idx_ref[...] == e
    o_ref[...] += jnp.where(mask, part, 0.0)


def kernel(hidden_states, weight_stacked, indices):
    idx = indices.astype(jnp.int32).reshape(B, 1)
    grid = (V // VT, D)
    return pl.pallas_call(
        _body,
        grid=grid,
        in_specs=[
            pl.BlockSpec((B, H), lambda v, e: (0, 0)),
            pl.BlockSpec((1, VT, H), lambda v, e: (e, v, 0)),
            pl.BlockSpec((B, 1), lambda v, e: (0, 0)),
        ],
        out_specs=pl.BlockSpec((B, VT), lambda v, e: (0, v)),
        out_shape=jax.ShapeDtypeStruct((B, V), jnp.float32),
    )(hidden_states, weight_stacked, idx)


# speedup vs baseline: 1.0094x; 1.0011x over previous
import jax
import jax.numpy as jnp
from jax.experimental import pallas as pl
from jax.experimental.pallas import tpu as pltpu

B = 64
H = 4096
V = 32000
D = 8
VT = 1280


def _body(x_ref, w_ref, idx_ref, o_ref):
    e = pl.program_id(1)

    @pl.when(e == 0)
    def _():
        o_ref[...] = jnp.zeros_like(o_ref)

    xb = x_ref[...]
    wb = w_ref[0]
    part = jax.lax.dot_general(
        xb, wb, (((1,), (1,)), ((), ())),
        preferred_element_type=jnp.float32)
    mask = idx_ref[...] == e
    o_ref[...] += jnp.where(mask, part, 0.0)


def kernel(hidden_states, weight_stacked, indices):
    idx = indices.astype(jnp.int32).reshape(B, 1)
    grid = (V // VT, D)
    return pl.pallas_call(
        _body,
        grid=grid,
        in_specs=[
            pl.BlockSpec((B, H), lambda v, e: (0, 0)),
            pl.BlockSpec((1, VT, H), lambda v, e: (e, v, 0)),
            pl.BlockSpec((B, 1), lambda v, e: (0, 0)),
        ],
        out_specs=pl.BlockSpec((B, VT), lambda v, e: (0, v)),
        out_shape=jax.ShapeDtypeStruct((B, V), jnp.float32),
        compiler_params=pltpu.CompilerParams(
            dimension_semantics=("parallel", "arbitrary")),
    )(hidden_states, weight_stacked, idx)
